# baseline (speedup 1.0000x reference)
import functools

import jax
import jax.numpy as jnp
from jax import lax
from jax.experimental import pallas as pl
from jax.experimental.pallas import tpu as pltpu
from jax.experimental.pallas import tpu_sc as plsc

B = 16
Q = 256
T = 16384
TH = T // 2
CHUNK = 128
NCHUNK = TH // CHUNK


def _sc_body(idx_hbm, out_hbm, idx_v, idxs_v, zeros_v, ones_v, sem_z, sem_s):
    b = lax.axis_index("s")
    half = lax.axis_index("c")
    t0 = half * TH
    base = b * (Q * T) + t0

    pltpu.sync_copy(idx_hbm.at[b, pl.ds(t0, TH)], idx_v)

    def zinit(u, _):
        zeros_v[pl.ds(u * 16, 16)] = jnp.zeros((16,), jnp.float32)
        return 0

    lax.fori_loop(0, TH // 16, zinit, 0)

    def oinit(u, _):
        ones_v[pl.ds(u * 16, 16)] = jnp.full((16,), 1.0, jnp.float32)
        return 0

    lax.fori_loop(0, CHUNK // 16, oinit, 0)

    def zgroup(g, _):
        def zfire(q, _):
            pltpu.make_async_copy(
                zeros_v, out_hbm.at[pl.ds(base + q * T, TH)], sem_z
            ).start()
            return 0

        lax.fori_loop(g * 32, (g + 1) * 32, zfire, 0)

        def zdrain(q, _):
            pltpu.make_async_copy(
                zeros_v, out_hbm.at[pl.ds(base, TH)], sem_z
            ).wait()
            return 0

        lax.fori_loop(0, 32, zdrain, 0)
        return 0

    lax.fori_loop(0, Q // 32, zgroup, 0)

    lane = lax.iota(jnp.int32, 16)

    def cchunk(j, _):
        def cvec(u, _):
            toff = j * CHUNK + u * 16
            v = idx_v[pl.ds(toff, 16)]
            idxs_v[j, pl.ds(u * 16, 16)] = v * T + (base + toff) + lane
            return 0

        lax.fori_loop(0, CHUNK // 16, cvec, 0)
        return 0

    lax.fori_loop(0, NCHUNK, cchunk, 0)

    def sfire(j, _):
        pltpu.make_async_copy(ones_v, out_hbm.at[idxs_v.at[j]], sem_s).start()
        return 0

    lax.fori_loop(0, NCHUNK, sfire, 0)

    def sdrain(j, _):
        pltpu.make_async_copy(ones_v, out_hbm.at[idxs_v.at[0]], sem_s).wait()
        return 0

    lax.fori_loop(0, NCHUNK, sdrain, 0)


@functools.partial(
    pl.kernel,
    out_type=jax.ShapeDtypeStruct((B * Q * T,), jnp.float32),
    mesh=plsc.VectorSubcoreMesh(core_axis_name="c", subcore_axis_name="s"),
    scratch_types=[
        pltpu.VMEM((TH,), jnp.int32),
        pltpu.VMEM((NCHUNK, CHUNK), jnp.int32),
        pltpu.VMEM((TH,), jnp.float32),
        pltpu.VMEM((CHUNK,), jnp.float32),
        pltpu.SemaphoreType.DMA,
        pltpu.SemaphoreType.DMA,
    ],
)
def _sc_onehot(idx_hbm, out_hbm, idx_v, idxs_v, zeros_v, ones_v, sem_z, sem_s):
    _sc_body(idx_hbm, out_hbm, idx_v, idxs_v, zeros_v, ones_v, sem_z, sem_s)


def kernel(in_snd_slice, quant_onehot):
    del quant_onehot
    idx = in_snd_slice.astype(jnp.int32)
    flat = _sc_onehot(idx)
    return flat.reshape(B, Q, T)
